# baseline (speedup 1.0000x reference)
import functools

import jax
import jax.numpy as jnp
from jax import lax
from jax.experimental import pallas as pl
from jax.experimental.pallas import tpu as pltpu
from jax.experimental.pallas import tpu_sc as plsc

VOCAB = 100000
D = 1024
NEW_ROWS = 702
TOTAL = 8192
TRASH = TOTAL
MAX_IDX = VOCAB + NEW_ROWS - 2

NW = 32
B_PER_W = TOTAL // NW
CHUNK = 16
NCHUNK = B_PER_W // CHUNK


def _body(x_hbm, orig_hbm, new_hbm, out_hbm,
          idx_v, lo_v, hi_v, plo_v, phi_v, buf_a0, buf_b0, buf_a1, buf_b1,
          ga_s0, gb_s0, sa_s0, sb_s0, ga_s1, gb_s1, sa_s1, sb_s1):
    wid = lax.axis_index("s") * 2 + lax.axis_index("c")
    base = wid * B_PER_W

    pltpu.sync_copy(x_hbm.at[pl.ds(base, B_PER_W)], idx_v)

    for i in range(B_PER_W // 16):
        v = idx_v[pl.ds(i * 16, 16)]
        v = jnp.maximum(v, 0)
        vc = jnp.minimum(v, MAX_IDX)
        is_hi = v >= VOCAB
        lo = jnp.minimum(v, VOCAB - 1)
        hi = jnp.where(is_hi, vc - (VOCAB - 1), 0)
        row = base + i * 16 + lax.iota(jnp.int32, 16)
        lo_v[pl.ds(i * 16, 16)] = lo
        hi_v[pl.ds(i * 16, 16)] = hi
        plo_v[i, :] = jnp.where(is_hi, TRASH, row)
        phi_v[i, :] = jnp.where(is_hi, row, TRASH)

    buf_a = (buf_a0, buf_a1)
    buf_b = (buf_b0, buf_b1)
    ga_sem = (ga_s0, ga_s1)
    gb_sem = (gb_s0, gb_s1)
    sa_sem = (sa_s0, sa_s1)
    sb_sem = (sb_s0, sb_s1)
    scat = [None, None]
    for j in range(NCHUNK):
        s = j % 2
        if scat[s] is not None:
            scat[s][0].wait()
            scat[s][1].wait()
        ga = pltpu.async_copy(orig_hbm.at[lo_v.at[pl.ds(j * CHUNK, CHUNK)]],
                              buf_a[s], ga_sem[s])
        gb = pltpu.async_copy(new_hbm.at[hi_v.at[pl.ds(j * CHUNK, CHUNK)]],
                              buf_b[s], gb_sem[s])
        ga.wait()
        gb.wait()
        sa = pltpu.async_copy(buf_a[s], out_hbm.at[plo_v.at[j]], sa_sem[s])
        sb = pltpu.async_copy(buf_b[s], out_hbm.at[phi_v.at[j]], sb_sem[s])
        scat[s] = (sa, sb)
    for s in range(2):
        if scat[s] is not None:
            scat[s][0].wait()
            scat[s][1].wait()


@jax.jit
def _gather(x_flat, original_weight, new_weight):
    mesh = plsc.VectorSubcoreMesh(core_axis_name="c", subcore_axis_name="s")
    k = functools.partial(
        pl.kernel,
        mesh=mesh,
        out_type=jax.ShapeDtypeStruct((TOTAL + 1, D), jnp.float32),
        scratch_types=[
            pltpu.VMEM((B_PER_W,), jnp.int32),
            pltpu.VMEM((B_PER_W,), jnp.int32),
            pltpu.VMEM((B_PER_W,), jnp.int32),
            pltpu.VMEM((NCHUNK, CHUNK), jnp.int32),
            pltpu.VMEM((NCHUNK, CHUNK), jnp.int32),
            pltpu.VMEM((CHUNK, D), jnp.float32),
            pltpu.VMEM((CHUNK, D), jnp.float32),
            pltpu.VMEM((CHUNK, D), jnp.float32),
            pltpu.VMEM((CHUNK, D), jnp.float32),
        ] + [pltpu.SemaphoreType.DMA] * 8,
    )(_body)
    return k(x_flat, original_weight, new_weight)


def kernel(x, original_weight, new_weight):
    out = _gather(x.reshape(-1), original_weight, new_weight)
    return out[:TOTAL].reshape(*x.shape, D)


# speedup vs baseline: 4.4279x; 4.4279x over previous
import functools

import jax
import jax.numpy as jnp
from jax import lax
from jax.experimental import pallas as pl
from jax.experimental.pallas import tpu as pltpu
from jax.experimental.pallas import tpu_sc as plsc

VOCAB = 100000
D = 1024
NEW_ROWS = 702
TOTAL = 8192
TRASH = TOTAL
MAX_IDX = VOCAB + NEW_ROWS - 2

NW = 32
B_PER_W = TOTAL // NW
CHUNK = 32
NCHUNK = B_PER_W // CHUNK
NGROUP = B_PER_W // 16
GPC = CHUNK // 16


def _body(x_hbm, orig_hbm, new_hbm, out_hbm,
          idx_v, lo_v, hi_v, plo_v, phi_v, buf0, buf1, fbuf,
          g_s0, g_s1, w_s0, w_s1, f_s):
    wid = lax.axis_index("s") * 2 + lax.axis_index("c")
    base = wid * B_PER_W

    pltpu.sync_copy(x_hbm.at[pl.ds(base, B_PER_W)], idx_v)

    cnts = []
    for i in range(NGROUP):
        v = idx_v[pl.ds(i * 16, 16)]
        v = jnp.maximum(v, 0)
        vc = jnp.minimum(v, MAX_IDX)
        is_hi = v >= VOCAB
        row = base + i * 16 + lax.iota(jnp.int32, 16)
        hi = jnp.where(is_hi, vc - (VOCAB - 1), 0)
        lo_v[pl.ds(i * 16, 16)] = jnp.minimum(v, VOCAB - 1)
        hi_v[i, :] = hi
        phi_v[i, :] = jnp.where(is_hi, row, TRASH)
        j, g = divmod(i, GPC)
        plo_v[j, pl.ds(g * 16, 16)] = jnp.where(is_hi, TRASH, row)
        acc = hi[0]
        for t in range(1, 16):
            acc = acc + hi[t]
        cnts.append(acc)

    bufs = (buf0, buf1)
    g_sem = (g_s0, g_s1)
    w_sem = (w_s0, w_s1)
    gath = [None, None]
    fired = [False, False]
    gath[0] = pltpu.async_copy(orig_hbm.at[lo_v.at[pl.ds(0, CHUNK)]],
                               bufs[0], g_sem[0])
    for j in range(NCHUNK):
        s = j % 2
        o = (j + 1) % 2
        ccnt = cnts[GPC * j]
        for g in range(1, GPC):
            ccnt = ccnt + cnts[GPC * j + g]
        if j + 1 < NCHUNK:
            if fired[o]:
                pltpu.make_async_copy(
                    bufs[o], out_hbm.at[pl.ds(base + (j - 1) * CHUNK, CHUNK)],
                    w_sem[o]).wait()
            gath[o] = pltpu.async_copy(
                orig_hbm.at[lo_v.at[pl.ds((j + 1) * CHUNK, CHUNK)]],
                bufs[o], g_sem[o])
        gath[s].wait()

        @pl.when(ccnt == 0)
        def _lin(s=s, j=j):
            pltpu.async_copy(bufs[s],
                             out_hbm.at[pl.ds(base + j * CHUNK, CHUNK)],
                             w_sem[s])

        @pl.when(ccnt != 0)
        def _idx(s=s, j=j):
            pltpu.async_copy(bufs[s], out_hbm.at[plo_v.at[j]], w_sem[s])

        fired[s] = True
    for s in range(2):
        if fired[s]:
            pltpu.make_async_copy(
                bufs[s], out_hbm.at[pl.ds(base, CHUNK)], w_sem[s]).wait()

    for i in range(NGROUP):
        @pl.when(cnts[i] > 0)
        def _fix(i=i):
            pltpu.async_copy(new_hbm.at[hi_v.at[i]], fbuf, f_s).wait()
            pltpu.async_copy(fbuf, out_hbm.at[phi_v.at[i]], f_s).wait()


@jax.jit
def _gather(x_flat, original_weight, new_weight):
    mesh = plsc.VectorSubcoreMesh(core_axis_name="c", subcore_axis_name="s")
    k = functools.partial(
        pl.kernel,
        mesh=mesh,
        out_type=jax.ShapeDtypeStruct((TOTAL + 1, D), jnp.float32),
        scratch_types=[
            pltpu.VMEM((B_PER_W,), jnp.int32),
            pltpu.VMEM((B_PER_W,), jnp.int32),
            pltpu.VMEM((NGROUP, 16), jnp.int32),
            pltpu.VMEM((NCHUNK, CHUNK), jnp.int32),
            pltpu.VMEM((NGROUP, 16), jnp.int32),
            pltpu.VMEM((CHUNK, D), jnp.float32),
            pltpu.VMEM((CHUNK, D), jnp.float32),
            pltpu.VMEM((16, D), jnp.float32),
        ] + [pltpu.SemaphoreType.DMA] * 5,
    )(_body)
    return k(x_flat, original_weight, new_weight)


def kernel(x, original_weight, new_weight):
    out = _gather(x.reshape(-1), original_weight, new_weight)
    return out[:TOTAL].reshape(*x.shape, D)


# speedup vs baseline: 4.4512x; 1.0053x over previous
import functools

import jax
import jax.numpy as jnp
from jax import lax
from jax.experimental import pallas as pl
from jax.experimental.pallas import tpu as pltpu
from jax.experimental.pallas import tpu_sc as plsc

VOCAB = 100000
D = 1024
NEW_ROWS = 702
TOTAL = 8192
MAX_IDX = VOCAB + NEW_ROWS - 2

NW = 32
B_PER_W = TOTAL // NW
CHUNK = 32
NCHUNK = B_PER_W // CHUNK
NGROUP = B_PER_W // 16
GPC = CHUNK // 16
DBLK = D // 16


def _body(x_hbm, orig_hbm, new_hbm, out_hbm,
          idx_v, lo_v, hi_v, buf0, buf1, fbuf,
          g_s0, g_s1, w_s0, w_s1, f_s):
    wid = lax.axis_index("s") * 2 + lax.axis_index("c")
    base = wid * B_PER_W

    pltpu.sync_copy(x_hbm.at[pl.ds(base, B_PER_W)], idx_v)

    cnts = []
    for i in range(NGROUP):
        v = idx_v[pl.ds(i * 16, 16)]
        v = jnp.maximum(v, 0)
        vc = jnp.minimum(v, MAX_IDX)
        is_hi = v >= VOCAB
        hi = jnp.where(is_hi, vc - (VOCAB - 1), 0)
        lo_v[pl.ds(i * 16, 16)] = jnp.minimum(v, VOCAB - 1)
        hi_v[i, :] = hi
        acc = hi[0]
        for t in range(1, 16):
            acc = acc + hi[t]
        cnts.append(acc)

    bufs = (buf0, buf1)
    g_sem = (g_s0, g_s1)
    w_sem = (w_s0, w_s1)
    gath = [None, None]
    fired = [False, False]
    gath[0] = pltpu.async_copy(orig_hbm.at[lo_v.at[pl.ds(0, CHUNK)]],
                               bufs[0], g_sem[0])
    for j in range(NCHUNK):
        s = j % 2
        o = (j + 1) % 2
        if j + 1 < NCHUNK:
            if fired[o]:
                pltpu.make_async_copy(
                    bufs[o], out_hbm.at[pl.ds(base + (j - 1) * CHUNK, CHUNK)],
                    w_sem[o]).wait()
            gath[o] = pltpu.async_copy(
                orig_hbm.at[lo_v.at[pl.ds((j + 1) * CHUNK, CHUNK)]],
                bufs[o], g_sem[o])
        gath[s].wait()

        for g in range(GPC):
            i = GPC * j + g

            @pl.when(cnts[i] > 0)
            def _patch(s=s, g=g, i=i):
                pltpu.async_copy(new_hbm.at[hi_v.at[i]], fbuf, f_s).wait()
                hv = hi_v[i, :]
                for t in range(16):
                    @pl.when(hv[t] > 0)
                    def _lane(s=s, g=g, t=t):
                        def _cp(q, _):
                            col = q * 16
                            bufs[s][g * 16 + t, pl.ds(col, 16)] = (
                                fbuf[t, pl.ds(col, 16)])
                            return 0
                        lax.fori_loop(0, DBLK, _cp, 0)

        pltpu.async_copy(bufs[s],
                         out_hbm.at[pl.ds(base + j * CHUNK, CHUNK)],
                         w_sem[s])
        fired[s] = True
    for s in range(2):
        if fired[s]:
            pltpu.make_async_copy(
                bufs[s], out_hbm.at[pl.ds(base, CHUNK)], w_sem[s]).wait()


@jax.jit
def _gather(x_flat, original_weight, new_weight):
    mesh = plsc.VectorSubcoreMesh(core_axis_name="c", subcore_axis_name="s")
    k = functools.partial(
        pl.kernel,
        mesh=mesh,
        out_type=jax.ShapeDtypeStruct((TOTAL, D), jnp.float32),
        scratch_types=[
            pltpu.VMEM((B_PER_W,), jnp.int32),
            pltpu.VMEM((B_PER_W,), jnp.int32),
            pltpu.VMEM((NGROUP, 16), jnp.int32),
            pltpu.VMEM((CHUNK, D), jnp.float32),
            pltpu.VMEM((CHUNK, D), jnp.float32),
            pltpu.VMEM((16, D), jnp.float32),
        ] + [pltpu.SemaphoreType.DMA] * 5,
    )(_body)
    return k(x_flat, original_weight, new_weight)


def kernel(x, original_weight, new_weight):
    out = _gather(x.reshape(-1), original_weight, new_weight)
    return out.reshape(*x.shape, D)
